# baseline (speedup 1.0000x reference)
import jax
import jax.numpy as jnp
from jax.experimental import pallas as pl
from jax.experimental.pallas import tpu as pltpu

_R = 8192
_C = 512
_W = 4
_BLK = 512


def _mae_body(*refs):
    p_refs = refs[:_W]
    t_refs = refs[_W:2 * _W]
    out_ref = refs[2 * _W]
    err_acc, cnt_acc = refs[2 * _W + 1], refs[2 * _W + 2]
    i = pl.program_id(0)

    @pl.when(i == 0)
    def _init():
        err_acc[...] = jnp.zeros_like(err_acc)
        cnt_acc[...] = jnp.zeros_like(cnt_acc)

    e = jnp.zeros((1, _C), jnp.float32)
    c = jnp.zeros((1, _C), jnp.float32)
    for p_ref, t_ref in zip(p_refs, t_refs):
        p = p_ref[...]
        t = t_ref[...]
        m = jnp.where(t > 0.0, 1.0, 0.0)
        e += jnp.sum(jnp.abs(p - t) * m, axis=0, keepdims=True)
        c += jnp.sum(m, axis=0, keepdims=True)
    err_acc[...] += e
    cnt_acc[...] += c

    @pl.when(i == pl.num_programs(0) - 1)
    def _fini():
        s = jnp.sum(err_acc[...])
        n = jnp.sum(cnt_acc[...])
        mae = s / jnp.maximum(n, 1.0)
        out_ref[0, 0] = jnp.where(n < 10.0, jnp.float32(-1.0), mae)


def kernel(pred, target):
    p = pred.reshape(_R, _C)
    t = target.reshape(_R, _C)
    steps = _R // _W // _BLK
    specs = [
        pl.BlockSpec((_BLK, _C), lambda i, w=w: (i + w * steps, 0))
        for w in range(_W)
    ]
    out = pl.pallas_call(
        _mae_body,
        grid=(steps,),
        in_specs=specs + specs,
        out_specs=pl.BlockSpec(memory_space=pltpu.SMEM),
        out_shape=jax.ShapeDtypeStruct((1, 1), jnp.float32),
        scratch_shapes=[
            pltpu.VMEM((1, _C), jnp.float32),
            pltpu.VMEM((1, _C), jnp.float32),
        ],
    )(*([p] * _W + [t] * _W))
    return out[0, 0]


# speedup vs baseline: 1.0199x; 1.0199x over previous
import jax
import jax.numpy as jnp
from jax.experimental import pallas as pl
from jax.experimental.pallas import tpu as pltpu

_R = 8192
_C = 512
_W = 4
_BLK = 512


def _mae_body(*refs):
    p_refs = refs[:_W]
    t_refs = refs[_W:2 * _W]
    out_ref = refs[2 * _W]
    err_acc, cnt_acc = refs[2 * _W + 1], refs[2 * _W + 2]
    i = pl.program_id(0)

    @pl.when(i == 0)
    def _init():
        err_acc[...] = jnp.zeros_like(err_acc)
        cnt_acc[...] = jnp.zeros_like(cnt_acc)

    e = jnp.zeros((1, _C), jnp.float32)
    c = jnp.zeros((1, _C), jnp.float32)
    for p_ref, t_ref in zip(p_refs, t_refs):
        p = p_ref[...]
        t = t_ref[...]
        valid = t > 0.0
        err = jnp.where(valid, jnp.abs(p - t), 0.0)
        cnt = jnp.where(valid, 1.0, 0.0)
        e += jnp.sum(err, axis=0, keepdims=True)
        c += jnp.sum(cnt, axis=0, keepdims=True)
    err_acc[...] += e
    cnt_acc[...] += c

    @pl.when(i == pl.num_programs(0) - 1)
    def _fini():
        s = jnp.sum(err_acc[...])
        n = jnp.sum(cnt_acc[...])
        mae = s / jnp.maximum(n, 1.0)
        out_ref[0, 0] = jnp.where(n < 10.0, jnp.float32(-1.0), mae)


def kernel(pred, target):
    p = pred.reshape(_R, _C)
    t = target.reshape(_R, _C)
    steps = _R // _W // _BLK
    specs = [
        pl.BlockSpec((_BLK, _C), lambda i, w=w: (i + w * steps, 0))
        for w in range(_W)
    ]
    out = pl.pallas_call(
        _mae_body,
        grid=(steps,),
        in_specs=specs + specs,
        out_specs=pl.BlockSpec(memory_space=pltpu.SMEM),
        out_shape=jax.ShapeDtypeStruct((1, 1), jnp.float32),
        scratch_shapes=[
            pltpu.VMEM((1, _C), jnp.float32),
            pltpu.VMEM((1, _C), jnp.float32),
        ],
    )(*([p] * _W + [t] * _W))
    return out[0, 0]
